# Initial kernel scaffold; baseline (speedup 1.0000x reference)
import jax
import jax.numpy as jnp
from jax.experimental import pallas as pl


def kernel(x1, x2, edge_index1, edge_index2, batch1, batch2, ddi_type, W_node, b_node, W_g1, b_g1, W_g2, b_g2, Q_pool, Q_inter, W_m1, b_m1, W_m2, b_m2, W_m3, b_m3, W_m4, b_m4):
    raise NotImplementedError("write your pallas kernel here")


# speedup vs baseline: 12.9191x; 12.9191x over previous
import functools

import jax
import jax.numpy as jnp
from jax import lax
from jax.experimental import pallas as pl
from jax.experimental.pallas import tpu as pltpu
from jax.experimental.pallas import tpu_sc as plsc

_N = 10000
_NP = 10240
_E = 320000
_D = 128
_P = 8
_B = 64
_NC = 2
_NS = 16
_EPT = _E // _NS
_RPT = _NP // _NS
_CH = 80
_NCH = _EPT // _CH
_IDXC = 2000


@functools.cache
def _sc_degree_fn():
    return functools.partial(
        pl.kernel,
        out_type=jax.ShapeDtypeStruct((_NC * _NS * _NP,), jnp.float32),
        scratch_types=[
            pltpu.VMEM((_NP,), jnp.float32),
            pltpu.VMEM((_IDXC,), jnp.int32),
        ],
        mesh=plsc.VectorSubcoreMesh(core_axis_name="c", subcore_axis_name="s",
                                    num_cores=_NC, num_subcores=_NS),
        compiler_params=pltpu.CompilerParams(needs_layout_passes=False),
    )(_sc_degree_body)


def _sc_degree_body(dsts, out, acc, idxb):
    c = lax.axis_index("c")
    s = lax.axis_index("s")
    z16 = jnp.zeros((16,), jnp.float32)
    ones16 = jnp.ones((16,), jnp.float32)

    def zero_body(i, _):
        acc[pl.ds(i * 16, 16)] = z16
        return 0
    lax.fori_loop(0, _NP // 16, zero_body, 0)

    base = c * _E + s * _EPT

    def chunk_body(ci, _):
        pltpu.sync_copy(dsts.at[pl.ds(base + ci * _IDXC, _IDXC)], idxb)

        def grp(j, _):
            v = idxb[pl.ds(j * 16, 16)]
            plsc.addupdate_scatter(acc, [v], ones16)
            return 0
        lax.fori_loop(0, _IDXC // 16, grp, 0)
        return 0
    lax.fori_loop(0, _EPT // _IDXC, chunk_body, 0)
    pltpu.sync_copy(acc, out.at[pl.ds((c * _NS + s) * _NP, _NP)])


@functools.cache
def _sc_edge_fn():
    return functools.partial(
        pl.kernel,
        out_type=jax.ShapeDtypeStruct((_NC, _NP, _D), jnp.float32),
        scratch_types=[
            pltpu.VMEM_SHARED((_NP, _D), jnp.float32),
            pltpu.VMEM((128, _D), jnp.float32),
            pltpu.VMEM((_CH,), jnp.int32),
            pltpu.VMEM((_CH,), jnp.int32),
            pltpu.VMEM((_CH, _D), jnp.float32),
            pltpu.SemaphoreType.DMA,
        ],
        mesh=plsc.VectorSubcoreMesh(core_axis_name="c", subcore_axis_name="s",
                                    num_cores=_NC, num_subcores=_NS),
        compiler_params=pltpu.CompilerParams(needs_layout_passes=False),
    )(_sc_edge_body)


def _sc_edge_body(ys, srcs, dsts, out, aggsp, zb, sidx, didx, rows, sem):
    c = lax.axis_index("c")
    s = lax.axis_index("s")
    z16 = jnp.zeros((16,), jnp.float32)

    def zfill(i, _):
        zb[i // 8, pl.ds((i % 8) * 16, 16)] = z16
        return 0
    lax.fori_loop(0, 128 * (_D // 16), zfill, 0)

    def zcopy(i, _):
        pltpu.sync_copy(zb, aggsp.at[pl.ds(s * _RPT + i * 128, 128)])
        return 0
    lax.fori_loop(0, _RPT // 128, zcopy, 0)
    plsc.subcore_barrier()

    base = c * _E + s * _EPT

    def ebody(i, _):
        off = base + i * _CH
        pltpu.sync_copy(srcs.at[pl.ds(off, _CH)], sidx)
        pltpu.sync_copy(dsts.at[pl.ds(off, _CH)], didx)
        pltpu.async_copy(ys.at[c].at[sidx], rows, sem).wait()
        pltpu.sync_copy(rows, aggsp.at[didx], add=True)
        return 0
    lax.fori_loop(0, _NCH, ebody, 0)
    plsc.subcore_barrier()
    pltpu.sync_copy(aggsp.at[pl.ds(s * _RPT, _RPT)],
                    out.at[c, pl.ds(s * _RPT, _RPT)])


_R = 512


def _inv_from_parts(dp):
    deg = jnp.sum(dp, axis=1) + 1.0
    return lax.rsqrt(jnp.maximum(deg, 1.0))


def _tc_pre_body(x_ref, dp_ref, w_ref, b_ref, y_ref):
    inv = _inv_from_parts(dp_ref[0])
    h = jnp.dot(x_ref[0], w_ref[...], preferred_element_type=jnp.float32, precision=lax.Precision.HIGHEST)
    y_ref[0] = (h + b_ref[...][None, :]) * inv[:, None]


def _tc_pre(xs, degp, W, b):
    return pl.pallas_call(
        _tc_pre_body,
        grid=(2, _NP // _R),
        in_specs=[
            pl.BlockSpec((1, _R, _D), lambda g, i: (g, i, 0)),
            pl.BlockSpec((1, _R, _NS), lambda g, i: (g, i, 0)),
            pl.BlockSpec((_D, _D), lambda g, i: (0, 0)),
            pl.BlockSpec((_D,), lambda g, i: (0,)),
        ],
        out_specs=pl.BlockSpec((1, _R, _D), lambda g, i: (g, i, 0)),
        out_shape=jax.ShapeDtypeStruct((2, _NP, _D), jnp.float32),
    )(xs, degp, W, b)


def _tc_post_body(s_ref, y_ref, dp_ref, w_ref, b_ref, h_ref, yn_ref):
    inv = _inv_from_parts(dp_ref[0])
    agg = (s_ref[0] + y_ref[0]) * inv[:, None]
    h = jnp.dot(agg, w_ref[...], preferred_element_type=jnp.float32, precision=lax.Precision.HIGHEST)
    h = jnp.maximum(h + b_ref[...][None, :], 0.0)
    h_ref[0] = h
    yn_ref[0] = h * inv[:, None]


def _tc_post(Ss, ys, degp, W, b):
    return pl.pallas_call(
        _tc_post_body,
        grid=(2, _NP // _R),
        in_specs=[
            pl.BlockSpec((1, _R, _D), lambda g, i: (g, i, 0)),
            pl.BlockSpec((1, _R, _D), lambda g, i: (g, i, 0)),
            pl.BlockSpec((1, _R, _NS), lambda g, i: (g, i, 0)),
            pl.BlockSpec((_D, _D), lambda g, i: (0, 0)),
            pl.BlockSpec((_D,), lambda g, i: (0,)),
        ],
        out_specs=[
            pl.BlockSpec((1, _R, _D), lambda g, i: (g, i, 0)),
            pl.BlockSpec((1, _R, _D), lambda g, i: (g, i, 0)),
        ],
        out_shape=[
            jax.ShapeDtypeStruct((2, _NP, _D), jnp.float32),
            jax.ShapeDtypeStruct((2, _NP, _D), jnp.float32),
        ],
    )(Ss, ys, degp, W, b)


def _tc_pool_body(h_ref, bt_ref, q_ref, po_ref, den_ref):
    i = pl.program_id(1)
    h = h_ref[0]
    s = lax.dot_general(h, q_ref[...], (((1,), (1,)), ((), ())),
                        preferred_element_type=jnp.float32, precision=lax.Precision.HIGHEST)
    e = jnp.exp(s)
    bt = bt_ref[0, 0]
    gids = lax.broadcasted_iota(jnp.int32, (_R, _B), 1)
    oh = (gids == bt[:, None]).astype(jnp.float32)
    den = lax.dot_general(oh, e, (((0,), (0,)), ((), ())),
                          preferred_element_type=jnp.float32, precision=lax.Precision.HIGHEST)

    @pl.when(i == 0)
    def _():
        den_ref[0] = den
        for pq in range(16):
            wh = h * e[:, pq][:, None]
            po_ref[0, pq] = lax.dot_general(
                oh, wh, (((0,), (0,)), ((), ())),
                preferred_element_type=jnp.float32, precision=lax.Precision.HIGHEST)

    @pl.when(i > 0)
    def _():
        den_ref[0] += den
        for pq in range(16):
            wh = h * e[:, pq][:, None]
            po_ref[0, pq] += lax.dot_general(
                oh, wh, (((0,), (0,)), ((), ())),
                preferred_element_type=jnp.float32, precision=lax.Precision.HIGHEST)


def _tc_pool(hs, bt3, Qcat):
    return pl.pallas_call(
        _tc_pool_body,
        grid=(2, _NP // _R),
        in_specs=[
            pl.BlockSpec((1, _R, _D), lambda g, i: (g, i, 0)),
            pl.BlockSpec((1, 1, _R), lambda g, i: (g * (_NP // _R) + i, 0, 0)),
            pl.BlockSpec((2 * _P, _D), lambda g, i: (0, 0)),
        ],
        out_specs=[
            pl.BlockSpec((1, 2 * _P, _B, _D), lambda g, i: (g, 0, 0, 0)),
            pl.BlockSpec((1, _B, 2 * _P), lambda g, i: (g, 0, 0)),
        ],
        out_shape=[
            jax.ShapeDtypeStruct((2, 2 * _P, _B, _D), jnp.float32),
            jax.ShapeDtypeStruct((2, _B, 2 * _P), jnp.float32),
        ],
    )(hs, bt3, Qcat)


_BP = _B * _P


def _tc_final_body(pi1_ref, pi2_ref, pp1_ref, pp2_ref,
                   di1_ref, di2_ref, dp1_ref, dp2_ref,
                   wm1_ref, bm1_ref, wm2_ref, bm2_ref,
                   wm3_ref, bm3_ref, wm4_ref, bm4_ref, out_ref):
    pi1 = pi1_ref[...] / jnp.maximum(di1_ref[...], 1e-9)
    pi2 = pi2_ref[...] / jnp.maximum(di2_ref[...], 1e-9)
    pp1 = pp1_ref[...] / jnp.maximum(dp1_ref[...], 1e-9)
    pp2 = pp2_ref[...] / jnp.maximum(dp2_ref[...], 1e-9)

    n1 = jnp.sqrt(jnp.sum(pp1 * pp1, axis=1, keepdims=True))
    p1n = pp1 / jnp.maximum(n1, 1e-12)
    n2 = jnp.sqrt(jnp.sum(pp2 * pp2, axis=1, keepdims=True))
    p2n = pp2 / jnp.maximum(n2, 1e-12)

    rr = lax.broadcasted_iota(jnp.int32, (_BP, _BP), 0)
    cc = lax.broadcasted_iota(jnp.int32, (_BP, _BP), 1)
    same = (rr // _P) == (cc // _P)
    neg = jnp.float32(-1e30)
    sc = jnp.float32(1.0) / jnp.sqrt(jnp.float32(_D))

    def blockdiag_softmax_matmul(a, b):
        f = lax.dot_general(a, b, (((1,), (1,)), ((), ())),
                            preferred_element_type=jnp.float32, precision=lax.Precision.HIGHEST) * sc
        f = jnp.where(same, f, neg)
        m = jnp.max(f, axis=1, keepdims=True)
        ex = jnp.exp(f - m)
        dn = jnp.sum(ex, axis=1, keepdims=True)
        attn = ex / dn
        return jnp.dot(attn, b, preferred_element_type=jnp.float32, precision=lax.Precision.HIGHEST)

    f1 = blockdiag_softmax_matmul(pi1, pi2)
    f2 = blockdiag_softmax_matmul(pi2, pi1)

    rb = lax.broadcasted_iota(jnp.int32, (_B, _BP), 0)
    cb = lax.broadcasted_iota(jnp.int32, (_B, _BP), 1)
    Mavg = jnp.where(rb == (cb // _P), jnp.float32(1.0 / _P), 0.0)
    f1m = jnp.dot(Mavg, f1, preferred_element_type=jnp.float32, precision=lax.Precision.HIGHEST)
    f2m = jnp.dot(Mavg, f2, preferred_element_type=jnp.float32, precision=lax.Precision.HIGHEST)

    def rownorm(x):
        mu = jnp.sum(x, axis=1, keepdims=True) / jnp.float32(_D)
        xm = x - mu
        var = jnp.sum(xm * xm, axis=1, keepdims=True) / jnp.float32(_D - 1)
        return xm * lax.rsqrt(var)

    inter1 = rownorm(f1m)
    inter2 = rownorm(f2m)

    fp = lax.dot_general(p1n, p2n, (((1,), (1,)), ((), ())),
                         preferred_element_type=jnp.float32, precision=lax.Precision.HIGHEST)
    fpm = jnp.where(same, fp, 0.0)
    ccq = lax.broadcasted_iota(jnp.int32, (_BP, _P), 0)
    ccc = lax.broadcasted_iota(jnp.int32, (_BP, _P), 1)
    C = ((ccq % _P) == ccc).astype(jnp.float32)
    FC = jnp.dot(fpm, C, preferred_element_type=jnp.float32, precision=lax.Precision.HIGHEST)

    w1 = wm1_ref[...]
    h = jnp.dot(inter1, w1[0:_D], preferred_element_type=jnp.float32, precision=lax.Precision.HIGHEST)
    h = h + jnp.dot(inter2, w1[_D:2 * _D], preferred_element_type=jnp.float32, precision=lax.Precision.HIGHEST)
    for p in range(_P):
        Ep = ((cb % _P) == p).astype(jnp.float32) * (rb == (cb // _P)).astype(jnp.float32)
        FCp = jnp.dot(Ep, FC, preferred_element_type=jnp.float32, precision=lax.Precision.HIGHEST)
        h = h + jnp.dot(FCp, w1[2 * _D + p * _P: 2 * _D + (p + 1) * _P],
                        preferred_element_type=jnp.float32, precision=lax.Precision.HIGHEST)
    h = h + bm1_ref[...][None, :]
    h = jnp.maximum(jnp.dot(h, wm2_ref[...], preferred_element_type=jnp.float32, precision=lax.Precision.HIGHEST)
                    + bm2_ref[...][None, :], 0.0)
    h = jnp.maximum(jnp.dot(h, wm3_ref[...], preferred_element_type=jnp.float32, precision=lax.Precision.HIGHEST)
                    + bm3_ref[...][None, :], 0.0)
    out_ref[...] = (jnp.dot(h, wm4_ref[...], preferred_element_type=jnp.float32, precision=lax.Precision.HIGHEST)
                    + bm4_ref[...][None, :])


def _tc_final(pi1, pi2, pp1, pp2, di1, di2, dp1, dp2,
              wm1, bm1, wm2, bm2, wm3, bm3, wm4, bm4):
    full = lambda s: pl.BlockSpec(s, lambda: tuple(0 for _ in s))
    args = [pi1, pi2, pp1, pp2, di1, di2, dp1, dp2,
            wm1, bm1, wm2, bm2, wm3, bm3, wm4, bm4]
    return pl.pallas_call(
        _tc_final_body,
        in_specs=[full(a.shape) for a in args],
        out_specs=full((_B, 1)),
        out_shape=jax.ShapeDtypeStruct((_B, 1), jnp.float32),
    )(*args)


def kernel(x1, x2, edge_index1, edge_index2, batch1, batch2, ddi_type,
           W_node, b_node, W_g1, b_g1, W_g2, b_g2, Q_pool, Q_inter,
           W_m1, b_m1, W_m2, b_m2, W_m3, b_m3, W_m4, b_m4):
    pad = _NP - _N
    xs = jnp.pad(jnp.stack([x1, x2]), ((0, 0), (0, pad), (0, 0)))
    srcs = jnp.concatenate([edge_index1[0], edge_index2[0]])
    dsts = jnp.concatenate([edge_index1[1], edge_index2[1]])
    bt3 = jnp.pad(jnp.stack([batch1, batch2]), ((0, 0), (0, pad)),
                  constant_values=-1).reshape(2 * (_NP // _R), 1, _R)
    Qcat = jnp.concatenate([Q_inter, Q_pool], axis=0)

    degp = _sc_degree_fn()(dsts).reshape(_NC, _NS, _NP).transpose(0, 2, 1)
    y0 = _tc_pre(xs, degp, W_node, b_node)
    S0 = _sc_edge_fn()(y0, srcs, dsts)
    h1, y1 = _tc_post(S0, y0, degp, W_g1, b_g1)
    del h1
    S1 = _sc_edge_fn()(y1, srcs, dsts)
    h2, _y2 = _tc_post(S1, y1, degp, W_g2, b_g2)

    pooled, den = _tc_pool(h2, bt3, Qcat)
    pb = pooled.transpose(0, 2, 1, 3)
    pi1 = pb[0, :, :_P, :].reshape(_BP, _D)
    pi2 = pb[1, :, :_P, :].reshape(_BP, _D)
    pp1 = pb[0, :, _P:, :].reshape(_BP, _D)
    pp2 = pb[1, :, _P:, :].reshape(_BP, _D)
    di1 = den[0, :, :_P].reshape(_BP, 1)
    di2 = den[1, :, :_P].reshape(_BP, 1)
    dp1 = den[0, :, _P:].reshape(_BP, 1)
    dp2 = den[1, :, _P:].reshape(_BP, 1)

    score = _tc_final(pi1, pi2, pp1, pp2, di1, di2, dp1, dp2,
                      W_m1, b_m1, W_m2, b_m2, W_m3, b_m3, W_m4, b_m4)
    return score.reshape(_B)


# speedup vs baseline: 14.3688x; 1.1122x over previous
import functools

import jax
import jax.numpy as jnp
from jax import lax
from jax.experimental import pallas as pl
from jax.experimental.pallas import tpu as pltpu
from jax.experimental.pallas import tpu_sc as plsc

_N = 10000
_NP = 10240
_E = 320000
_D = 128
_P = 8
_B = 64
_NC = 2
_NS = 16
_RPT = _NP // _NS
_CH = 128
_CPT = 160
_CPQ = 40
_EPTP = _CH * _CPT
_EP = _EPTP * _NS


@functools.cache
def _sc_degree_fn():
    return functools.partial(
        pl.kernel,
        out_type=jax.ShapeDtypeStruct((_NC * _NS * _NP,), jnp.float32),
        scratch_types=[
            pltpu.VMEM((_NP,), jnp.float32),
            pltpu.VMEM((16, _CH), jnp.int32),
        ],
        mesh=plsc.VectorSubcoreMesh(core_axis_name="c", subcore_axis_name="s",
                                    num_cores=_NC, num_subcores=_NS),
        compiler_params=pltpu.CompilerParams(needs_layout_passes=False),
    )(_sc_degree_body)


def _sc_degree_body(dst2, out, acc, idxb):
    c = lax.axis_index("c")
    s = lax.axis_index("s")
    z16 = jnp.zeros((16,), jnp.float32)
    ones16 = jnp.ones((16,), jnp.float32)

    def zero_body(i, _):
        acc[pl.ds(i * 16, 16)] = z16
        return 0
    lax.fori_loop(0, _NP // 16, zero_body, 0)

    base = (c * _NS + s) * _CPT

    def chunk_body(ci, _):
        pltpu.sync_copy(dst2.at[pl.ds(base + ci * 16, 16)], idxb)

        def grp(j, _):
            v = idxb[j // 8, pl.ds((j % 8) * 16, 16)]
            plsc.addupdate_scatter(acc, [v], ones16)
            return 0
        lax.fori_loop(0, 16 * (_CH // 16), grp, 0)
        return 0
    lax.fori_loop(0, _CPT // 16, chunk_body, 0)
    pltpu.sync_copy(acc, out.at[pl.ds((c * _NS + s) * _NP, _NP)])


@functools.cache
def _sc_edge_fn():
    return functools.partial(
        pl.kernel,
        out_type=jax.ShapeDtypeStruct((_NC, _NP, _D), jnp.float32),
        scratch_types=[
            pltpu.VMEM_SHARED((_NP, _D), jnp.float32),
            pltpu.VMEM((8, _D), jnp.float32),
            pltpu.VMEM((_CPQ, _CH), jnp.int32),
            pltpu.VMEM((_CPQ, _CH), jnp.int32),
            pltpu.VMEM((_CH, _D), jnp.float32),
            pltpu.VMEM((_CH, _D), jnp.float32),
            pltpu.SemaphoreType.DMA,
            pltpu.SemaphoreType.DMA,
        ],
        mesh=plsc.VectorSubcoreMesh(core_axis_name="c", subcore_axis_name="s",
                                    num_cores=_NC, num_subcores=_NS),
        compiler_params=pltpu.CompilerParams(needs_layout_passes=False),
    )(_sc_edge_body)


def _sc_edge_body(ys, src2, dst2, out, aggsp, zb, sidx, didx,
                  rows0, rows1, sem0, sem1):
    c = lax.axis_index("c")
    s = lax.axis_index("s")
    z16 = jnp.zeros((16,), jnp.float32)
    rb = (c * _NS + s) * _CPT

    def zfill(i, _):
        zb[i // 8, pl.ds((i % 8) * 16, 16)] = z16
        return 0
    lax.fori_loop(0, 8 * (_D // 16), zfill, 0)

    def zcopy(i, _):
        pltpu.sync_copy(zb, aggsp.at[pl.ds(s * _RPT + i * 8, 8)])
        return 0
    lax.fori_loop(0, _RPT // 8, zcopy, 0)
    plsc.subcore_barrier()

    def quarter(q, _):
        pltpu.sync_copy(src2.at[pl.ds(rb + q * _CPQ, _CPQ)], sidx)
        pltpu.sync_copy(dst2.at[pl.ds(rb + q * _CPQ, _CPQ)], didx)
        pltpu.async_copy(ys.at[c].at[sidx.at[0]], rows0, sem0)

        def ebody(k, _):
            i0 = 2 * k
            h1 = pltpu.async_copy(ys.at[c].at[sidx.at[i0 + 1]], rows1, sem1)
            pltpu.make_async_copy(ys.at[c].at[sidx.at[i0]],
                                  rows0, sem0).wait()
            pltpu.sync_copy(rows0, aggsp.at[didx.at[i0]], add=True)

            @pl.when(k + 1 < _CPQ // 2)
            def _():
                pltpu.async_copy(ys.at[c].at[sidx.at[i0 + 2]], rows0, sem0)
            h1.wait()
            pltpu.sync_copy(rows1, aggsp.at[didx.at[i0 + 1]], add=True)
            return 0
        lax.fori_loop(0, _CPQ // 2, ebody, 0)
        return 0
    lax.fori_loop(0, _CPT // _CPQ, quarter, 0)
    plsc.subcore_barrier()
    pltpu.sync_copy(aggsp.at[pl.ds(s * _RPT, _RPT)],
                    out.at[c, pl.ds(s * _RPT, _RPT)])


_R = 512


def _inv_from_parts(dp):
    deg = jnp.sum(dp, axis=1) + 1.0
    return lax.rsqrt(jnp.maximum(deg, 1.0))


def _tc_pre_body(x_ref, dp_ref, w_ref, b_ref, y_ref):
    inv = _inv_from_parts(dp_ref[0])
    h = jnp.dot(x_ref[0], w_ref[...], preferred_element_type=jnp.float32, precision=lax.Precision.HIGHEST)
    y_ref[0] = (h + b_ref[...][None, :]) * inv[:, None]


def _tc_pre(xs, degp, W, b):
    return pl.pallas_call(
        _tc_pre_body,
        grid=(2, _NP // _R),
        in_specs=[
            pl.BlockSpec((1, _R, _D), lambda g, i: (g, i, 0)),
            pl.BlockSpec((1, _R, _NS), lambda g, i: (g, i, 0)),
            pl.BlockSpec((_D, _D), lambda g, i: (0, 0)),
            pl.BlockSpec((_D,), lambda g, i: (0,)),
        ],
        out_specs=pl.BlockSpec((1, _R, _D), lambda g, i: (g, i, 0)),
        out_shape=jax.ShapeDtypeStruct((2, _NP, _D), jnp.float32),
    )(xs, degp, W, b)


def _tc_post_body(s_ref, y_ref, dp_ref, w_ref, b_ref, h_ref, yn_ref):
    inv = _inv_from_parts(dp_ref[0])
    agg = (s_ref[0] + y_ref[0]) * inv[:, None]
    h = jnp.dot(agg, w_ref[...], preferred_element_type=jnp.float32, precision=lax.Precision.HIGHEST)
    h = jnp.maximum(h + b_ref[...][None, :], 0.0)
    h_ref[0] = h
    yn_ref[0] = h * inv[:, None]


def _tc_post(Ss, ys, degp, W, b):
    return pl.pallas_call(
        _tc_post_body,
        grid=(2, _NP // _R),
        in_specs=[
            pl.BlockSpec((1, _R, _D), lambda g, i: (g, i, 0)),
            pl.BlockSpec((1, _R, _D), lambda g, i: (g, i, 0)),
            pl.BlockSpec((1, _R, _NS), lambda g, i: (g, i, 0)),
            pl.BlockSpec((_D, _D), lambda g, i: (0, 0)),
            pl.BlockSpec((_D,), lambda g, i: (0,)),
        ],
        out_specs=[
            pl.BlockSpec((1, _R, _D), lambda g, i: (g, i, 0)),
            pl.BlockSpec((1, _R, _D), lambda g, i: (g, i, 0)),
        ],
        out_shape=[
            jax.ShapeDtypeStruct((2, _NP, _D), jnp.float32),
            jax.ShapeDtypeStruct((2, _NP, _D), jnp.float32),
        ],
    )(Ss, ys, degp, W, b)


def _tc_pool_body(h_ref, bt_ref, q_ref, po_ref, den_ref):
    i = pl.program_id(1)
    h = h_ref[0]
    s = lax.dot_general(h, q_ref[...], (((1,), (1,)), ((), ())),
                        preferred_element_type=jnp.float32, precision=lax.Precision.HIGHEST)
    e = jnp.exp(s)
    bt = bt_ref[0, 0]
    gids = lax.broadcasted_iota(jnp.int32, (_R, _B), 1)
    oh = (gids == bt[:, None]).astype(jnp.float32)
    den = lax.dot_general(oh, e, (((0,), (0,)), ((), ())),
                          preferred_element_type=jnp.float32, precision=lax.Precision.HIGHEST)

    @pl.when(i == 0)
    def _():
        den_ref[0] = den
        for pq in range(16):
            wh = h * e[:, pq][:, None]
            po_ref[0, pq] = lax.dot_general(
                oh, wh, (((0,), (0,)), ((), ())),
                preferred_element_type=jnp.float32, precision=lax.Precision.HIGHEST)

    @pl.when(i > 0)
    def _():
        den_ref[0] += den
        for pq in range(16):
            wh = h * e[:, pq][:, None]
            po_ref[0, pq] += lax.dot_general(
                oh, wh, (((0,), (0,)), ((), ())),
                preferred_element_type=jnp.float32, precision=lax.Precision.HIGHEST)


def _tc_pool(hs, bt3, Qcat):
    return pl.pallas_call(
        _tc_pool_body,
        grid=(2, _NP // _R),
        in_specs=[
            pl.BlockSpec((1, _R, _D), lambda g, i: (g, i, 0)),
            pl.BlockSpec((1, 1, _R), lambda g, i: (g * (_NP // _R) + i, 0, 0)),
            pl.BlockSpec((2 * _P, _D), lambda g, i: (0, 0)),
        ],
        out_specs=[
            pl.BlockSpec((1, 2 * _P, _B, _D), lambda g, i: (g, 0, 0, 0)),
            pl.BlockSpec((1, _B, 2 * _P), lambda g, i: (g, 0, 0)),
        ],
        out_shape=[
            jax.ShapeDtypeStruct((2, 2 * _P, _B, _D), jnp.float32),
            jax.ShapeDtypeStruct((2, _B, 2 * _P), jnp.float32),
        ],
    )(hs, bt3, Qcat)


_BP = _B * _P


def _tc_final_body(pi1_ref, pi2_ref, pp1_ref, pp2_ref,
                   di1_ref, di2_ref, dp1_ref, dp2_ref,
                   wm1_ref, bm1_ref, wm2_ref, bm2_ref,
                   wm3_ref, bm3_ref, wm4_ref, bm4_ref, out_ref):
    pi1 = pi1_ref[...] / jnp.maximum(di1_ref[...], 1e-9)
    pi2 = pi2_ref[...] / jnp.maximum(di2_ref[...], 1e-9)
    pp1 = pp1_ref[...] / jnp.maximum(dp1_ref[...], 1e-9)
    pp2 = pp2_ref[...] / jnp.maximum(dp2_ref[...], 1e-9)

    n1 = jnp.sqrt(jnp.sum(pp1 * pp1, axis=1, keepdims=True))
    p1n = pp1 / jnp.maximum(n1, 1e-12)
    n2 = jnp.sqrt(jnp.sum(pp2 * pp2, axis=1, keepdims=True))
    p2n = pp2 / jnp.maximum(n2, 1e-12)

    rr = lax.broadcasted_iota(jnp.int32, (_BP, _BP), 0)
    cc = lax.broadcasted_iota(jnp.int32, (_BP, _BP), 1)
    same = (rr // _P) == (cc // _P)
    neg = jnp.float32(-1e30)
    sc = jnp.float32(1.0) / jnp.sqrt(jnp.float32(_D))

    def blockdiag_softmax_matmul(a, b):
        f = lax.dot_general(a, b, (((1,), (1,)), ((), ())),
                            preferred_element_type=jnp.float32, precision=lax.Precision.HIGHEST) * sc
        f = jnp.where(same, f, neg)
        m = jnp.max(f, axis=1, keepdims=True)
        ex = jnp.exp(f - m)
        dn = jnp.sum(ex, axis=1, keepdims=True)
        attn = ex / dn
        return jnp.dot(attn, b, preferred_element_type=jnp.float32, precision=lax.Precision.HIGHEST)

    f1 = blockdiag_softmax_matmul(pi1, pi2)
    f2 = blockdiag_softmax_matmul(pi2, pi1)

    rb = lax.broadcasted_iota(jnp.int32, (_B, _BP), 0)
    cb = lax.broadcasted_iota(jnp.int32, (_B, _BP), 1)
    Mavg = jnp.where(rb == (cb // _P), jnp.float32(1.0 / _P), 0.0)
    f1m = jnp.dot(Mavg, f1, preferred_element_type=jnp.float32, precision=lax.Precision.HIGHEST)
    f2m = jnp.dot(Mavg, f2, preferred_element_type=jnp.float32, precision=lax.Precision.HIGHEST)

    def rownorm(x):
        mu = jnp.sum(x, axis=1, keepdims=True) / jnp.float32(_D)
        xm = x - mu
        var = jnp.sum(xm * xm, axis=1, keepdims=True) / jnp.float32(_D - 1)
        return xm * lax.rsqrt(var)

    inter1 = rownorm(f1m)
    inter2 = rownorm(f2m)

    fp = lax.dot_general(p1n, p2n, (((1,), (1,)), ((), ())),
                         preferred_element_type=jnp.float32, precision=lax.Precision.HIGHEST)
    fpm = jnp.where(same, fp, 0.0)
    ccq = lax.broadcasted_iota(jnp.int32, (_BP, _P), 0)
    ccc = lax.broadcasted_iota(jnp.int32, (_BP, _P), 1)
    C = ((ccq % _P) == ccc).astype(jnp.float32)
    FC = jnp.dot(fpm, C, preferred_element_type=jnp.float32, precision=lax.Precision.HIGHEST)

    w1 = wm1_ref[...]
    h = jnp.dot(inter1, w1[0:_D], preferred_element_type=jnp.float32, precision=lax.Precision.HIGHEST)
    h = h + jnp.dot(inter2, w1[_D:2 * _D], preferred_element_type=jnp.float32, precision=lax.Precision.HIGHEST)
    for p in range(_P):
        Ep = ((cb % _P) == p).astype(jnp.float32) * (rb == (cb // _P)).astype(jnp.float32)
        FCp = jnp.dot(Ep, FC, preferred_element_type=jnp.float32, precision=lax.Precision.HIGHEST)
        h = h + jnp.dot(FCp, w1[2 * _D + p * _P: 2 * _D + (p + 1) * _P],
                        preferred_element_type=jnp.float32, precision=lax.Precision.HIGHEST)
    h = h + bm1_ref[...][None, :]
    h = jnp.maximum(jnp.dot(h, wm2_ref[...], preferred_element_type=jnp.float32, precision=lax.Precision.HIGHEST)
                    + bm2_ref[...][None, :], 0.0)
    h = jnp.maximum(jnp.dot(h, wm3_ref[...], preferred_element_type=jnp.float32, precision=lax.Precision.HIGHEST)
                    + bm3_ref[...][None, :], 0.0)
    out_ref[...] = (jnp.dot(h, wm4_ref[...], preferred_element_type=jnp.float32, precision=lax.Precision.HIGHEST)
                    + bm4_ref[...][None, :])


def _tc_final(pi1, pi2, pp1, pp2, di1, di2, dp1, dp2,
              wm1, bm1, wm2, bm2, wm3, bm3, wm4, bm4):
    full = lambda s: pl.BlockSpec(s, lambda: tuple(0 for _ in s))
    args = [pi1, pi2, pp1, pp2, di1, di2, dp1, dp2,
            wm1, bm1, wm2, bm2, wm3, bm3, wm4, bm4]
    return pl.pallas_call(
        _tc_final_body,
        in_specs=[full(a.shape) for a in args],
        out_specs=full((_B, 1)),
        out_shape=jax.ShapeDtypeStruct((_B, 1), jnp.float32),
    )(*args)


def kernel(x1, x2, edge_index1, edge_index2, batch1, batch2, ddi_type,
           W_node, b_node, W_g1, b_g1, W_g2, b_g2, Q_pool, Q_inter,
           W_m1, b_m1, W_m2, b_m2, W_m3, b_m3, W_m4, b_m4):
    pad = _NP - _N
    epad = _EP - _E
    xs = jnp.pad(jnp.stack([x1, x2]), ((0, 0), (0, pad), (0, 0)))
    fill = jnp.full((epad,), _NP - 1, jnp.int32)
    src2 = jnp.concatenate([edge_index1[0], fill, edge_index2[0], fill]
                           ).reshape(_NC * _NS * _CPT, _CH)
    dst2 = jnp.concatenate([edge_index1[1], fill, edge_index2[1], fill]
                           ).reshape(_NC * _NS * _CPT, _CH)
    bt3 = jnp.pad(jnp.stack([batch1, batch2]), ((0, 0), (0, pad)),
                  constant_values=-1).reshape(2 * (_NP // _R), 1, _R)
    Qcat = jnp.concatenate([Q_inter, Q_pool], axis=0)

    degp = _sc_degree_fn()(dst2).reshape(_NC, _NS, _NP).transpose(0, 2, 1)
    y0 = _tc_pre(xs, degp, W_node, b_node)
    S0 = _sc_edge_fn()(y0, src2, dst2)
    h1, y1 = _tc_post(S0, y0, degp, W_g1, b_g1)
    del h1
    S1 = _sc_edge_fn()(y1, src2, dst2)
    h2, _y2 = _tc_post(S1, y1, degp, W_g2, b_g2)

    pooled, den = _tc_pool(h2, bt3, Qcat)
    pb = pooled.transpose(0, 2, 1, 3)
    pi1 = pb[0, :, :_P, :].reshape(_BP, _D)
    pi2 = pb[1, :, :_P, :].reshape(_BP, _D)
    pp1 = pb[0, :, _P:, :].reshape(_BP, _D)
    pp2 = pb[1, :, _P:, :].reshape(_BP, _D)
    di1 = den[0, :, :_P].reshape(_BP, 1)
    di2 = den[1, :, :_P].reshape(_BP, 1)
    dp1 = den[0, :, _P:].reshape(_BP, 1)
    dp2 = den[1, :, _P:].reshape(_BP, 1)

    score = _tc_final(pi1, pi2, pp1, pp2, di1, di2, dp1, dp2,
                      W_m1, b_m1, W_m2, b_m2, W_m3, b_m3, W_m4, b_m4)
    return score.reshape(_B)
